# Initial kernel scaffold; baseline (speedup 1.0000x reference)
import jax
import jax.numpy as jnp
from jax.experimental import pallas as pl


def kernel(x1, x2, feaure1, feature2, W0, b0, gamma0, beta0, W1, b1, gamma1, beta1):
    raise NotImplementedError("write your pallas kernel here")


# speedup vs baseline: 15.8108x; 15.8108x over previous
import functools

import jax
import jax.numpy as jnp
from jax.experimental import pallas as pl


def _bar(v):
    return jax.lax.bitcast_convert_type(
        jax.lax.bitcast_convert_type(v, jnp.int32) + jnp.int32(0), jnp.float32)


def _bdot(a, b):
    return jnp.dot(a.astype(jnp.bfloat16), b.astype(jnp.bfloat16),
                   preferred_element_type=jnp.float32)


def _k1_body(x1_ref, x2t_ref, f1_ref, f2_ref, w0at_ref, w0bt_ref, b0_ref,
             h1_ref, s1_ref, ss1_ref, *, n2):
    x1 = x1_ref[0]
    x2t = x2t_ref[0]
    tq = x1.shape[0]

    ab = _bdot(x1, x2t)
    x1s = _bar(x1 * x1)
    a2 = _bar(_bar(x1s[:, 0:1] + x1s[:, 1:2]) + x1s[:, 2:3])
    x2s = _bar(x2t * x2t)
    b2 = _bar(_bar(x2s[0:1, :] + x2s[1:2, :]) + x2s[2:3, :])
    dist = _bar(_bar(_bar(-2.0 * ab) + a2) + b2)

    iota = jax.lax.broadcasted_iota(jnp.int32, (tq, n2), 1)
    d = dist
    wsp = jnp.zeros((tq, n2), jnp.float32)
    rsum = jnp.zeros((tq, 1), jnp.float32)
    for _ in range(3):
        m = jnp.min(d, axis=1, keepdims=True)
        idx = jnp.min(jnp.where(d == m, iota, n2), axis=1, keepdims=True)
        onehot = iota == idx
        r = 1.0 / (m + 1e-8)
        wsp = wsp + jnp.where(onehot, r, 0.0)
        rsum = rsum + r
        d = jnp.where(onehot, jnp.inf, d)
    wsp = wsp / rsum

    interp = jnp.dot(wsp, f2_ref[0], preferred_element_type=jnp.float32,
                     precision=jax.lax.Precision.HIGHEST)
    h1 = (_bdot(f1_ref[0], w0at_ref[...])
          + _bdot(interp, w0bt_ref[...])
          + b0_ref[...])
    h1_ref[0] = h1

    @pl.when((pl.program_id(0) == 0) & (pl.program_id(1) == 0))
    def _():
        s1_ref[...] = jnp.zeros_like(s1_ref)
        ss1_ref[...] = jnp.zeros_like(ss1_ref)

    s1_ref[...] += jnp.sum(h1, axis=0, keepdims=True)
    ss1_ref[...] += jnp.sum(h1 * h1, axis=0, keepdims=True)


def _k2_body(h1_ref, s1_ref, ss1_ref, g0_ref, bt0_ref, w1t_ref, b1_ref,
             h2_ref, s2_ref, ss2_ref, *, count):
    mean = s1_ref[...] / count
    var = ss1_ref[...] / count - mean * mean
    rstd = jax.lax.rsqrt(var + 1e-5)
    a1 = jnp.maximum((h1_ref[0] - mean) * (rstd * g0_ref[...]) + bt0_ref[...],
                     0.0)
    h2 = _bdot(a1, w1t_ref[...]) + b1_ref[...]
    h2_ref[0] = h2

    @pl.when((pl.program_id(0) == 0) & (pl.program_id(1) == 0))
    def _():
        s2_ref[...] = jnp.zeros_like(s2_ref)
        ss2_ref[...] = jnp.zeros_like(ss2_ref)

    s2_ref[...] += jnp.sum(h2, axis=0, keepdims=True)
    ss2_ref[...] += jnp.sum(h2 * h2, axis=0, keepdims=True)


def _k3_body(h2_ref, s2_ref, ss2_ref, g1_ref, bt1_ref, out_ref, *, count):
    mean = s2_ref[...] / count
    var = ss2_ref[...] / count - mean * mean
    rstd = jax.lax.rsqrt(var + 1e-5)
    out_ref[0] = jnp.maximum(
        (h2_ref[0] - mean) * (rstd * g1_ref[...]) + bt1_ref[...], 0.0)


def kernel(x1, x2, feaure1, feature2, W0, b0, gamma0, beta0, W1, b1, gamma1, beta1):
    B, N1, _ = x1.shape
    N2 = x2.shape[1]
    C1 = feaure1.shape[-1]
    C2 = feature2.shape[-1]
    H1 = W0.shape[0]
    H2 = W1.shape[0]
    count = float(B * N1)

    TQ1 = min(256, N1)
    TQ2 = min(512, N1)
    TQ3 = min(1024, N1)
    nb1 = N1 // TQ1
    nb2 = N1 // TQ2
    nb3 = N1 // TQ3

    x2t = jnp.swapaxes(x2, 1, 2)
    w0t = jnp.transpose(W0)
    w0at, w0bt = w0t[:C1], w0t[C1:]
    w1t = jnp.transpose(W1)
    b0r = b0.reshape(1, H1)
    g0r = gamma0.reshape(1, H1)
    bt0r = beta0.reshape(1, H1)
    b1r = b1.reshape(1, H2)
    g1r = gamma1.reshape(1, H2)
    bt1r = beta1.reshape(1, H2)

    rep = lambda shape: pl.BlockSpec(shape, lambda b, i: (0,) * len(shape))
    per_b = lambda shape: pl.BlockSpec(shape, lambda b, i: (b, 0, 0))
    tiled = lambda shape: pl.BlockSpec(shape, lambda b, i: (b, i, 0))

    h1, s1, ss1 = pl.pallas_call(
        functools.partial(_k1_body, n2=N2),
        grid=(B, nb1),
        in_specs=[
            tiled((1, TQ1, 3)),
            per_b((1, 3, N2)),
            tiled((1, TQ1, C1)),
            per_b((1, N2, C2)),
            rep((C1, H1)),
            rep((C2, H1)),
            rep((1, H1)),
        ],
        out_specs=[
            tiled((1, TQ1, H1)),
            rep((1, H1)),
            rep((1, H1)),
        ],
        out_shape=[
            jax.ShapeDtypeStruct((B, N1, H1), jnp.float32),
            jax.ShapeDtypeStruct((1, H1), jnp.float32),
            jax.ShapeDtypeStruct((1, H1), jnp.float32),
        ],
    )(x1, x2t, feaure1, feature2, w0at, w0bt, b0r)

    h2, s2, ss2 = pl.pallas_call(
        functools.partial(_k2_body, count=count),
        grid=(B, nb2),
        in_specs=[
            tiled((1, TQ2, H1)),
            rep((1, H1)),
            rep((1, H1)),
            rep((1, H1)),
            rep((1, H1)),
            rep((H1, H2)),
            rep((1, H2)),
        ],
        out_specs=[
            tiled((1, TQ2, H2)),
            rep((1, H2)),
            rep((1, H2)),
        ],
        out_shape=[
            jax.ShapeDtypeStruct((B, N1, H2), jnp.float32),
            jax.ShapeDtypeStruct((1, H2), jnp.float32),
            jax.ShapeDtypeStruct((1, H2), jnp.float32),
        ],
    )(h1, s1, ss1, g0r, bt0r, w1t, b1r)

    out = pl.pallas_call(
        functools.partial(_k3_body, count=count),
        grid=(B, nb3),
        in_specs=[
            tiled((1, TQ3, H2)),
            rep((1, H2)),
            rep((1, H2)),
            rep((1, H2)),
            rep((1, H2)),
        ],
        out_specs=tiled((1, TQ3, H2)),
        out_shape=jax.ShapeDtypeStruct((B, N1, H2), jnp.float32),
    )(h2, s2, ss2, g1r, bt1r)

    return out
